# Initial kernel scaffold; baseline (speedup 1.0000x reference)
import functools

import jax
import jax.numpy as jnp
from jax import lax
from jax.experimental import pallas as pl
from jax.experimental.pallas import tpu as pltpu
from jax.experimental.pallas import tpu_sc as plsc

L = 16
NC = 2
NS = 16
NW = NC * NS

VOCAB = 2000
DIM = 8

TOTAL = 16384 * 50
PER_W = TOTAL // NW
CHUNK = 5120
N_CHUNKS = PER_W // CHUNK


@functools.partial(
    pl.kernel,
    out_type=jax.ShapeDtypeStruct((TOTAL * DIM,), jnp.float32),
    mesh=plsc.VectorSubcoreMesh(core_axis_name="c", subcore_axis_name="s"),
    scratch_types=[
        pltpu.VMEM((VOCAB * DIM,), jnp.float32),
        pltpu.VMEM((CHUNK,), jnp.int32),
        pltpu.VMEM((CHUNK * DIM,), jnp.float32),
    ],
)
def _emb_lookup(ids_hbm, table_hbm, out_hbm, table_v, idx_v, out_v):
    wid = lax.axis_index("s") * NC + lax.axis_index("c")
    pltpu.sync_copy(table_hbm, table_v)

    iota_d = lax.iota(jnp.int32, L) * DIM

    def chunk_body(c, carry):
        base = wid * PER_W + c * CHUNK
        pltpu.sync_copy(ids_hbm.at[pl.ds(base, CHUNK)], idx_v)

        def gather_body(i, carry2):
            ids16 = idx_v[pl.ds(i * L, L)]
            rb = ids16 * DIM
            sb = i * (L * DIM) + iota_d
            for j in range(DIM):
                col = plsc.load_gather(table_v, [rb + j])
                plsc.store_scatter(out_v, [sb + j], col)
            return carry2

        lax.fori_loop(0, CHUNK // L, gather_body, 0)
        pltpu.sync_copy(out_v, out_hbm.at[pl.ds(base * DIM, CHUNK * DIM)])
        return carry

    lax.fori_loop(0, N_CHUNKS, chunk_body, 0)


def kernel(ids, base_embedding):
    b, s = ids.shape
    v, d = base_embedding.shape
    ids_flat = ids.reshape(-1).astype(jnp.int32)
    table_flat = base_embedding.reshape(-1).astype(jnp.float32)
    out = _emb_lookup(ids_flat, table_flat)
    return out.reshape(b, s, d)


# speedup vs baseline: 6.1483x; 6.1483x over previous
import functools

import jax
import jax.numpy as jnp
from jax import lax
from jax.experimental import pallas as pl
from jax.experimental.pallas import tpu as pltpu
from jax.experimental.pallas import tpu_sc as plsc

L = 16
NC = 2
NS = 16
NW = NC * NS

VOCAB = 2000
DIM = 8

TOTAL = 16384 * 50
PER_W = TOTAL // NW
CHUNK = 5120
N_CHUNKS = PER_W // CHUNK


@functools.partial(
    pl.kernel,
    out_type=jax.ShapeDtypeStruct((TOTAL * DIM,), jnp.float32),
    mesh=plsc.VectorSubcoreMesh(core_axis_name="c", subcore_axis_name="s"),
    compiler_params=pltpu.CompilerParams(needs_layout_passes=False),
    scratch_types=[
        pltpu.VMEM((VOCAB * DIM,), jnp.float32),
        pltpu.VMEM((CHUNK,), jnp.int32),
        pltpu.VMEM((CHUNK * DIM,), jnp.float32),
    ],
)
def _emb_lookup(ids_hbm, table_hbm, out_hbm, table_v, idx_v, out_v):
    wid = lax.axis_index("s") * NC + lax.axis_index("c")
    pltpu.sync_copy(table_hbm, table_v)

    iota_d = lax.iota(jnp.int32, L) * DIM

    def chunk_body(c, carry):
        base = wid * PER_W + c * CHUNK
        pltpu.sync_copy(ids_hbm.at[pl.ds(base, CHUNK)], idx_v)

        def gather_body(i, carry2):
            ids16 = idx_v[pl.ds(i * L, L)]
            rb = ids16 * DIM
            sb = i * (L * DIM) + iota_d
            for j in range(DIM):
                col = plsc.load_gather(table_v, [rb + j])
                plsc.store_scatter(out_v, [sb + j], col)
            return carry2

        lax.fori_loop(0, CHUNK // L, gather_body, 0)
        pltpu.sync_copy(out_v, out_hbm.at[pl.ds(base * DIM, CHUNK * DIM)])
        return carry

    lax.fori_loop(0, N_CHUNKS, chunk_body, 0)


def kernel(ids, base_embedding):
    b, s = ids.shape
    v, d = base_embedding.shape
    ids_flat = ids.reshape(-1).astype(jnp.int32)
    table_flat = base_embedding.reshape(-1).astype(jnp.float32)
    out = _emb_lookup(ids_flat, table_flat)
    return out.reshape(b, s, d)


# speedup vs baseline: 16.3172x; 2.6539x over previous
import functools

import jax
import jax.numpy as jnp
from jax import lax
from jax.experimental import pallas as pl
from jax.experimental.pallas import tpu as pltpu
from jax.experimental.pallas import tpu_sc as plsc

L = 16
NC = 2
NS = 16
NW = NC * NS

VOCAB = 2000
DIM = 8
B = 16384
S = 50
BT = B // 128

ROWS_W = B // NW
ROWS_C = 128
N_CHUNKS = ROWS_W // ROWS_C
CHUNK = ROWS_C * S


@functools.partial(
    pl.kernel,
    out_type=jax.ShapeDtypeStruct((S, BT, DIM, 128), jnp.float32),
    mesh=plsc.VectorSubcoreMesh(core_axis_name="c", subcore_axis_name="s"),
    compiler_params=pltpu.CompilerParams(needs_layout_passes=False),
    scratch_types=[
        pltpu.VMEM((VOCAB * DIM,), jnp.float32),
        pltpu.VMEM((CHUNK,), jnp.int32),
        pltpu.VMEM((S, 1, DIM, 128), jnp.float32),
    ],
)
def _emb_lookup(ids_hbm, table_hbm, out_hbm, table_v, idx_v, out_v):
    wid = lax.axis_index("s") * NC + lax.axis_index("c")
    pltpu.sync_copy(table_hbm, table_v)

    iota = lax.iota(jnp.int32, L)
    jvecs = [jnp.full((L,), j * 128, jnp.int32) for j in range(DIM)]

    def chunk_body(c, carry):
        bt = wid * N_CHUNKS + c
        pltpu.sync_copy(ids_hbm.at[pl.ds(bt * CHUNK, CHUNK)], idx_v)

        def gather_body(i, carry2):
            ids16 = idx_v[pl.ds(i * L, L)]
            rb = ids16 * DIM
            p = i * L + iota
            b_l = p // S
            s_i = p - b_l * S
            base = s_i * (DIM * 128) + b_l
            for j in range(DIM):
                col = plsc.load_gather(table_v, [rb + j])
                plsc.store_scatter(out_v, [s_i, jnp.zeros((L,), jnp.int32),
                                           jnp.full((L,), j, jnp.int32), b_l], col)
            return carry2

        lax.fori_loop(0, CHUNK // L, gather_body, 0)
        pltpu.sync_copy(out_v, out_hbm.at[:, pl.ds(bt, 1)])
        return carry

    lax.fori_loop(0, N_CHUNKS, chunk_body, 0)


def kernel(ids, base_embedding):
    ids_flat = ids.reshape(-1).astype(jnp.int32)
    table_flat = base_embedding.reshape(-1).astype(jnp.float32)
    out4 = _emb_lookup(ids_flat, table_flat)
    return out4.transpose(1, 3, 0, 2).reshape(B, S, DIM)


# speedup vs baseline: 24.1523x; 1.4802x over previous
import functools

import jax
import jax.numpy as jnp
from jax import lax
from jax.experimental import pallas as pl
from jax.experimental.pallas import tpu as pltpu
from jax.experimental.pallas import tpu_sc as plsc

L = 16
NC = 2
NS = 16
NW = NC * NS

VOCAB = 2000
DIM = 8
B = 16384
S = 50
BT = B // 128

ROWS_W = B // NW
ROWS_C = 128
N_CHUNKS = ROWS_W // ROWS_C
CHUNK = ROWS_C * S


@functools.partial(
    pl.kernel,
    out_type=jax.ShapeDtypeStruct((S, BT, DIM, 128), jnp.float32),
    mesh=plsc.VectorSubcoreMesh(core_axis_name="c", subcore_axis_name="s"),
    compiler_params=pltpu.CompilerParams(needs_layout_passes=False),
    scratch_types=[
        pltpu.VMEM((VOCAB * DIM,), jnp.float32),
        pltpu.VMEM((CHUNK,), jnp.int32),
        pltpu.VMEM((S, 1, DIM, 128), jnp.float32),
    ],
)
def _emb_lookup(ids_hbm, table_hbm, out_hbm, table_v, idx_v, out_v):
    wid = lax.axis_index("s") * NC + lax.axis_index("c")
    pltpu.sync_copy(table_hbm, table_v)

    iota_s = lax.iota(jnp.int32, L) * S

    def chunk_body(c, carry):
        bt = wid * N_CHUNKS + c
        pltpu.sync_copy(ids_hbm.at[pl.ds(bt * CHUNK, CHUNK)], idx_v)

        def s_body(s, carry2):
            for g in range(ROWS_C // L):
                ids16 = plsc.load_gather(idx_v, [iota_s + (g * L * S) + s])
                rb = ids16 * DIM
                for j in range(DIM):
                    col = plsc.load_gather(table_v, [rb + j])
                    out_v[s, 0, j, pl.ds(g * L, L)] = col
            return carry2

        lax.fori_loop(0, S, s_body, 0)
        pltpu.sync_copy(out_v, out_hbm.at[:, pl.ds(bt, 1)])
        return carry

    lax.fori_loop(0, N_CHUNKS, chunk_body, 0)


def kernel(ids, base_embedding):
    ids_flat = ids.reshape(-1).astype(jnp.int32)
    table_flat = base_embedding.reshape(-1).astype(jnp.float32)
    out4 = _emb_lookup(ids_flat, table_flat)
    return out4.transpose(1, 3, 0, 2).reshape(B, S, DIM)


# speedup vs baseline: 24.8818x; 1.0302x over previous
import functools

import jax
import jax.numpy as jnp
from jax import lax
from jax.experimental import pallas as pl
from jax.experimental.pallas import tpu as pltpu
from jax.experimental.pallas import tpu_sc as plsc

L = 16
NC = 2
NS = 16
NW = NC * NS

VOCAB = 2000
DIM = 8
B = 16384
S = 50
BT = B // 128

ROWS_W = B // NW
ROWS_C = 128
N_CHUNKS = ROWS_W // ROWS_C
CHUNK = ROWS_C * S


@functools.partial(
    pl.kernel,
    out_type=jax.ShapeDtypeStruct((S, BT, DIM, 128), jnp.float32),
    mesh=plsc.VectorSubcoreMesh(core_axis_name="c", subcore_axis_name="s"),
    compiler_params=pltpu.CompilerParams(needs_layout_passes=False),
    scratch_types=[
        pltpu.VMEM((VOCAB * DIM,), jnp.float32),
        pltpu.VMEM((CHUNK,), jnp.int32),
        pltpu.VMEM((S, 1, DIM, 128), jnp.float32),
        pltpu.VMEM((S, 1, DIM, 128), jnp.float32),
        pltpu.SemaphoreType.DMA,
        pltpu.SemaphoreType.DMA,
        pltpu.SemaphoreType.DMA,
    ],
)
def _emb_lookup(ids_hbm, table_hbm, out_hbm, table_v, idx_v,
                out_v0, out_v1, sem_t, sem_o0, sem_o1):
    wid = lax.axis_index("s") * NC + lax.axis_index("c")
    tbl_h = pltpu.async_copy(table_hbm, table_v, sem_t)

    iota_s = lax.iota(jnp.int32, L) * S
    out_bufs = (out_v0, out_v1)
    out_sems = (sem_o0, sem_o1)
    out_handles = [None, None]

    def make_s_body(out_v):
        def s_body(s, carry2):
            for g in range(ROWS_C // L):
                ids16 = plsc.load_gather(idx_v, [iota_s + (g * L * S) + s])
                rb = ids16 * DIM
                for j in range(DIM):
                    col = plsc.load_gather(table_v, [rb + j])
                    out_v[s, 0, j, pl.ds(g * L, L)] = col
            return carry2
        return s_body

    for c in range(N_CHUNKS):
        bt = wid * N_CHUNKS + c
        buf = c % 2
        pltpu.sync_copy(ids_hbm.at[pl.ds(bt * CHUNK, CHUNK)], idx_v)
        if c == 0:
            tbl_h.wait()
        if out_handles[buf] is not None:
            out_handles[buf].wait()
        lax.fori_loop(0, S, make_s_body(out_bufs[buf]), 0)
        out_handles[buf] = pltpu.async_copy(
            out_bufs[buf], out_hbm.at[:, pl.ds(bt, 1)], out_sems[buf])

    out_handles[0].wait()
    out_handles[1].wait()


def kernel(ids, base_embedding):
    ids_flat = ids.reshape(-1).astype(jnp.int32)
    table_flat = base_embedding.reshape(-1).astype(jnp.float32)
    out4 = _emb_lookup(ids_flat, table_flat)
    return out4.transpose(1, 3, 0, 2).reshape(B, S, DIM)


# speedup vs baseline: 46.7342x; 1.8783x over previous
import functools

import jax
import jax.numpy as jnp
from jax import lax
from jax.experimental import pallas as pl
from jax.experimental.pallas import tpu as pltpu
from jax.experimental.pallas import tpu_sc as plsc

L = 16
NC = 2
NS = 16
NW = NC * NS

VOCAB = 2000
DIM = 8
B = 16384
S = 50
BT = B // 128

ROWS_W = B // NW
ROWS_C = 128
N_CHUNKS = ROWS_W // ROWS_C
CHUNK = ROWS_C * S


@functools.partial(
    pl.kernel,
    out_type=jax.ShapeDtypeStruct((S, BT, DIM, 128), jnp.float32),
    mesh=plsc.VectorSubcoreMesh(core_axis_name="c", subcore_axis_name="s"),
    compiler_params=pltpu.CompilerParams(needs_layout_passes=False),
    scratch_types=[
        pltpu.VMEM((VOCAB * DIM,), jnp.float32),
        pltpu.VMEM((CHUNK,), jnp.int32),
        pltpu.VMEM((S, 1, DIM, 128), jnp.float32),
        pltpu.VMEM((S, 1, DIM, 128), jnp.float32),
        pltpu.SemaphoreType.DMA,
        pltpu.SemaphoreType.DMA,
        pltpu.SemaphoreType.DMA,
    ],
)
def _emb_lookup(ids_hbm, table_hbm, out_hbm, table_v, idx_v,
                out_v0, out_v1, sem_t, sem_o0, sem_o1):
    wid = lax.axis_index("s") * NC + lax.axis_index("c")
    tbl_h = pltpu.async_copy(table_hbm, table_v, sem_t)

    iota_s = lax.iota(jnp.int32, L) * S
    out_bufs = (out_v0, out_v1)
    out_sems = (sem_o0, sem_o1)
    out_handles = [None, None]

    NG = ROWS_C // L

    def make_s_body(out_v):
        def s_body(s, carry2):
            idsv = [plsc.load_gather(idx_v, [iota_s + (g * L * S) + s])
                    for g in range(NG)]
            rbs = [v * DIM for v in idsv]
            cols = [[plsc.load_gather(table_v, [rbs[g] + j])
                     for j in range(DIM)] for g in range(NG)]
            for g in range(NG):
                for j in range(DIM):
                    out_v[s, 0, j, pl.ds(g * L, L)] = cols[g][j]
            return carry2
        return s_body

    for c in range(N_CHUNKS):
        bt = wid * N_CHUNKS + c
        buf = c % 2
        pltpu.sync_copy(ids_hbm.at[pl.ds(bt * CHUNK, CHUNK)], idx_v)
        if c == 0:
            tbl_h.wait()
        if out_handles[buf] is not None:
            out_handles[buf].wait()
        lax.fori_loop(0, S, make_s_body(out_bufs[buf]), 0)
        out_handles[buf] = pltpu.async_copy(
            out_bufs[buf], out_hbm.at[:, pl.ds(bt, 1)], out_sems[buf])

    out_handles[0].wait()
    out_handles[1].wait()


def kernel(ids, base_embedding):
    ids_flat = ids.reshape(-1).astype(jnp.int32)
    table_flat = base_embedding.reshape(-1).astype(jnp.float32)
    out4 = _emb_lookup(ids_flat, table_flat)
    return out4.transpose(1, 3, 0, 2).reshape(B, S, DIM)
